# baseline (speedup 1.0000x reference)
import jax
import jax.numpy as jnp
from jax.experimental import pallas as pl
from jax.experimental.pallas import tpu as pltpu
from jax.experimental.pallas import tpu_sc as plsc

_K = 8192
_C = 256
_TN = 512
_TK = 1024
_COMMIT = 0.25


_SEG = (0, 2736, 5472, _K)


def _dist_argmin_kernel(x_ref, xsq_ref, w_ref, wsq_ref, amin_ref, dmin_ref):
    xb = x_ref[...]
    xb2 = xb + xb
    xsq = xsq_ref[...]
    lane = jax.lax.broadcasted_iota(jnp.int32, (1, _TK), 1)

    run_val = [None] * 3
    run_idx = [None] * 3

    def update(c, d, cidx):
        if run_val[c] is None:
            run_val[c] = d
            run_idx[c] = jnp.broadcast_to(cidx, (_TN, _TK))
        else:
            upd = d < run_val[c]
            run_idx[c] = jnp.where(upd, cidx, run_idx[c])
            run_val[c] = jnp.where(upd, d, run_val[c])

    seg_res = []

    def finish(c):
        gval = jnp.min(run_val[c], axis=1, keepdims=True)
        gidx = jnp.min(jnp.where(run_val[c] == gval, run_idx[c], _K),
                       axis=1, keepdims=True)
        seg_res.append((gval, gidx))
        run_val[c] = run_idx[c] = None

    last_chunk = [max(j for j in range(_K // _TK)
                      if j * _TK < _SEG[c + 1]) for c in range(3)]
    for j in range(_K // _TK):
        w_chunk = w_ref[j * _TK:(j + 1) * _TK, :]
        m2 = jax.lax.dot_general(
            xb2, w_chunk, (((1,), (1,)), ((), ())),
            preferred_element_type=jnp.float32)
        wsq_chunk = wsq_ref[:, j * _TK:(j + 1) * _TK]
        d = (xsq + wsq_chunk) - m2
        cidx = lane + j * _TK
        lo, hi = j * _TK, (j + 1) * _TK
        for c in range(3):
            slo, shi = _SEG[c], _SEG[c + 1]
            if hi <= slo or lo >= shi:
                continue
            if slo <= lo and hi <= shi:
                update(c, d, cidx)
            else:
                inlane = (cidx >= slo) & (cidx < shi)
                update(c, jnp.where(inlane, d, jnp.inf), cidx)
        for c in range(3):
            if last_chunk[c] == j:
                finish(c)

    b = None
    idx = None
    vpick = None
    for gval, gidx in seg_res:
        if b is None:
            b = gval.astype(jnp.bfloat16).astype(jnp.float32)
            idx = gidx
            vpick = gval
        else:
            upd = gval < b
            idx = jnp.where(upd, gidx, idx)
            vpick = jnp.where(upd, gval, vpick)
            b = jnp.where(upd, gval.astype(jnp.bfloat16).astype(jnp.float32), b)

    amin_ref[...] = idx
    dmin_ref[...] = vpick


def _dist_argmin(x_bf, xsq, w_bf, wsq):
    n = x_bf.shape[0]
    grid = (n // _TN,)
    return pl.pallas_call(
        _dist_argmin_kernel,
        grid=grid,
        in_specs=[
            pl.BlockSpec((_TN, _C), lambda i: (i, 0)),
            pl.BlockSpec((_TN, 1), lambda i: (i, 0)),
            pl.BlockSpec((_K, _C), lambda i: (0, 0)),
            pl.BlockSpec((1, _K), lambda i: (0, 0)),
        ],
        out_specs=[
            pl.BlockSpec((_TN, 1), lambda i: (i, 0)),
            pl.BlockSpec((_TN, 1), lambda i: (i, 0)),
        ],
        out_shape=[
            jax.ShapeDtypeStruct((n, 1), jnp.int32),
            jax.ShapeDtypeStruct((n, 1), jnp.float32),
        ],
    )(x_bf, xsq, w_bf, wsq)


def _sc_gather_rows(weight, idx):
    n = idx.shape[0]
    window = 128
    idx2 = idx.reshape(1, n)
    mesh = plsc.VectorSubcoreMesh(core_axis_name="core",
                                  subcore_axis_name="subcore")

    @pl.kernel(out_type=jax.ShapeDtypeStruct((n, _C), weight.dtype), mesh=mesh)
    def gather_kernel(w_hbm, i_hbm, o_hbm):
        def body(i_vmem, o_vmem):
            pltpu.sync_copy(w_hbm.at[i_vmem.at[0]], o_vmem)

        pltpu.emit_pipeline(
            body,
            grid=(n // window,),
            in_specs=[pl.BlockSpec((1, window), index_map=lambda i: (0, i))],
            out_specs=[pl.BlockSpec((window, _C), index_map=lambda i: (i, 0))],
            core_axis_name=("core", "subcore"),
            dimension_semantics=(pltpu.PARALLEL,),
        )(i_hbm, o_hbm)

    return gather_kernel(weight, idx2)


def kernel(x, weight):
    b, c, h, w = x.shape
    x_flat = jnp.transpose(x, (0, 2, 3, 1)).reshape(-1, c)
    xsq = jnp.sum(x_flat ** 2, axis=1, keepdims=True)
    wsq = jnp.sum(weight ** 2, axis=1, keepdims=True).T
    x_bf = x_flat.astype(jnp.bfloat16)
    w_bf = weight.astype(jnp.bfloat16)

    n = x_flat.shape[0]
    half = n // 2
    bh = b // 2
    zs, dmins = [], []
    for lo, hi in ((0, half), (half, n)):
        amin, dmin = _dist_argmin(x_bf[lo:hi], xsq[lo:hi], w_bf, wsq)
        zq = _sc_gather_rows(weight, amin.reshape(-1))
        zs.append(jnp.transpose(zq.reshape(bh, h, w, c), (0, 3, 1, 2)))
        dmins.append(dmin)
    z_q = jnp.concatenate(zs, axis=0)
    codebook_loss = (jnp.sum(dmins[0]) + jnp.sum(dmins[1])) / (b * c * h * w)
    commitment_loss = _COMMIT * codebook_loss
    return z_q, commitment_loss, codebook_loss


# speedup vs baseline: 1.1409x; 1.1409x over previous
import jax
import jax.numpy as jnp
from jax.experimental import pallas as pl
from jax.experimental.pallas import tpu as pltpu
from jax.experimental.pallas import tpu_sc as plsc

_K = 8192
_C = 256
_TN = 512
_TK = 1024
_COMMIT = 0.25


_SEG = (0, 2736, 5472, _K)


def _dist_argmin_kernel(x_ref, xsq_ref, w_ref, wsq_ref, amin_ref, dmin_ref):
    xb = x_ref[...]
    xb2 = (xb + xb).astype(jnp.bfloat16)
    xsq = xsq_ref[...]
    lane = jax.lax.broadcasted_iota(jnp.int32, (1, _TK), 1)

    run_val = [None] * 3
    run_idx = [None] * 3

    def update(c, d, cidx):
        if run_val[c] is None:
            run_val[c] = d
            run_idx[c] = jnp.broadcast_to(cidx, (_TN, _TK))
        else:
            upd = d < run_val[c]
            run_idx[c] = jnp.where(upd, cidx, run_idx[c])
            run_val[c] = jnp.where(upd, d, run_val[c])

    seg_res = []

    def finish(c):
        gval = jnp.min(run_val[c], axis=1, keepdims=True)
        gidx = jnp.min(jnp.where(run_val[c] == gval, run_idx[c], _K),
                       axis=1, keepdims=True)
        seg_res.append((gval, gidx))
        run_val[c] = run_idx[c] = None

    def chunk_d(j):
        w_chunk = w_ref[j * _TK:(j + 1) * _TK, :]
        m2 = jax.lax.dot_general(
            xb2, w_chunk, (((1,), (1,)), ((), ())),
            preferred_element_type=jnp.float32)
        wsq_chunk = wsq_ref[:, j * _TK:(j + 1) * _TK]
        d = (xsq + wsq_chunk) - m2
        return d, lane + j * _TK

    last_chunk = [max(j for j in range(_K // _TK)
                      if j * _TK < _SEG[c + 1]) for c in range(3)]
    straddle = {j for j in range(_K // _TK)
                for s in _SEG[1:3] if j * _TK < s < (j + 1) * _TK}
    j = 0
    while j < _K // _TK:
        if (j not in straddle and j + 1 < _K // _TK and j + 1 not in straddle
                and last_chunk.count(j) == 0):
            d0, i0 = chunk_d(j)
            d1, i1 = chunk_d(j + 1)
            pm = d1 < d0
            pv = jnp.where(pm, d1, d0)
            pi = jnp.where(pm, i1, i0)
            c = next(c for c in range(3)
                     if _SEG[c] <= j * _TK < _SEG[c + 1])
            update(c, pv, pi)
            for cc in range(3):
                if last_chunk[cc] == j + 1:
                    finish(cc)
            j += 2
            continue
        d, cidx = chunk_d(j)
        lo, hi = j * _TK, (j + 1) * _TK
        for c in range(3):
            slo, shi = _SEG[c], _SEG[c + 1]
            if hi <= slo or lo >= shi:
                continue
            if slo <= lo and hi <= shi:
                update(c, d, cidx)
            else:
                inlane = (cidx >= slo) & (cidx < shi)
                update(c, jnp.where(inlane, d, jnp.inf), cidx)
        for c in range(3):
            if last_chunk[c] == j:
                finish(c)
        j += 1

    b = None
    idx = None
    vpick = None
    for gval, gidx in seg_res:
        if b is None:
            b = gval.astype(jnp.bfloat16).astype(jnp.float32)
            idx = gidx
            vpick = gval
        else:
            upd = gval < b
            idx = jnp.where(upd, gidx, idx)
            vpick = jnp.where(upd, gval, vpick)
            b = jnp.where(upd, gval.astype(jnp.bfloat16).astype(jnp.float32), b)

    amin_ref[...] = idx
    dmin_ref[...] = vpick


def _dist_argmin(x_bf, xsq, w_bf, wsq):
    n = x_bf.shape[0]
    grid = (n // _TN,)
    return pl.pallas_call(
        _dist_argmin_kernel,
        grid=grid,
        in_specs=[
            pl.BlockSpec((_TN, _C), lambda i: (i, 0)),
            pl.BlockSpec((_TN, 1), lambda i: (i, 0)),
            pl.BlockSpec((_K, _C), lambda i: (0, 0)),
            pl.BlockSpec((1, _K), lambda i: (0, 0)),
        ],
        out_specs=[
            pl.BlockSpec((_TN, 1), lambda i: (i, 0)),
            pl.BlockSpec((_TN, 1), lambda i: (i, 0)),
        ],
        out_shape=[
            jax.ShapeDtypeStruct((n, 1), jnp.int32),
            jax.ShapeDtypeStruct((n, 1), jnp.float32),
        ],
    )(x_bf, xsq, w_bf, wsq)


def _sc_gather_rows(weight, idx):
    n = idx.shape[0]
    window = 128
    idx2 = idx.reshape(1, n)
    mesh = plsc.VectorSubcoreMesh(core_axis_name="core",
                                  subcore_axis_name="subcore")

    @pl.kernel(out_type=jax.ShapeDtypeStruct((n, _C), weight.dtype), mesh=mesh)
    def gather_kernel(w_hbm, i_hbm, o_hbm):
        def body(i_vmem, o_vmem):
            pltpu.sync_copy(w_hbm.at[i_vmem.at[0]], o_vmem)

        pltpu.emit_pipeline(
            body,
            grid=(n // window,),
            in_specs=[pl.BlockSpec((1, window), index_map=lambda i: (0, i))],
            out_specs=[pl.BlockSpec((window, _C), index_map=lambda i: (i, 0))],
            core_axis_name=("core", "subcore"),
            dimension_semantics=(pltpu.PARALLEL,),
        )(i_hbm, o_hbm)

    return gather_kernel(weight, idx2)


def kernel(x, weight):
    b, c, h, w = x.shape
    x_flat = jnp.transpose(x, (0, 2, 3, 1)).reshape(-1, c)
    xsq = jnp.sum(x_flat ** 2, axis=1, keepdims=True)
    wsq = jnp.sum(weight ** 2, axis=1, keepdims=True).T
    amin, dmin = _dist_argmin(x_flat, xsq, weight.astype(jnp.bfloat16), wsq)
    zq_flat = _sc_gather_rows(weight, amin.reshape(-1))
    z_q = jnp.transpose(zq_flat.reshape(b, h, w, c), (0, 3, 1, 2))
    codebook_loss = jnp.sum(dmin) / (b * c * h * w)
    commitment_loss = _COMMIT * codebook_loss
    return z_q, commitment_loss, codebook_loss
